# baseline (speedup 1.0000x reference)
import functools

import jax
import jax.numpy as jnp
from jax import lax
from jax.experimental import pallas as pl
from jax.experimental.pallas import tpu as pltpu
from jax.experimental.pallas import tpu_sc as plsc

_ROWS = 8192
_COLS = 2048
_NW = 32
_UNROLL = 8
_ROWS_PER_TILE = _ROWS // _NW
_PIECE_ROWS = 8
_NPIECE = _ROWS_PER_TILE // _PIECE_ROWS
_VPR = _COLS // 16


def _sc_hist_call(logits2d, targets2d):
    mesh = plsc.VectorSubcoreMesh(core_axis_name="c", subcore_axis_name="s")

    @functools.partial(
        pl.kernel,
        out_type=jax.ShapeDtypeStruct((_NW, _UNROLL, 512), jnp.float32),
        mesh=mesh,
        compiler_params=pltpu.CompilerParams(
            needs_layout_passes=False, use_tc_tiling_on_sc=True),
        scratch_types=[
            pltpu.VMEM((2 * _PIECE_ROWS, _COLS), jnp.float32),
            pltpu.VMEM((2 * _PIECE_ROWS, _COLS), jnp.float32),
        ] + [pltpu.VMEM((512,), jnp.float32) for _ in range(_UNROLL)] + [
            pltpu.SemaphoreType.DMA((2,)),
            pltpu.SemaphoreType.DMA((2,)),
        ],
    )
    def sc_ece(x_hbm, t_hbm, out_hbm, xbuf, tbuf, *rest):
        hists = rest[:_UNROLL]
        xsem, tsem = rest[_UNROLL:]
        wid = lax.axis_index("s") * 2 + lax.axis_index("c")
        row_base = wid * _ROWS_PER_TILE

        zeros16 = jnp.zeros((16,), jnp.float32)
        for h in hists:
            for k in range(32):
                h[pl.ds(16 * k, 16)] = zeros16

        lane_off = lax.iota(jnp.int32, 16) * 16
        hpak = [h.at[pl.ds(0, 256)] for h in hists]
        hcnf = [h.at[pl.ds(256, 256)] for h in hists]

        def start_piece(p, slot):
            r0 = pl.multiple_of(row_base + p * _PIECE_ROWS, _PIECE_ROWS)
            dst = pl.ds(slot * _PIECE_ROWS, _PIECE_ROWS)
            pltpu.async_copy(x_hbm.at[pl.ds(r0, _PIECE_ROWS), :],
                             xbuf.at[dst, :], xsem.at[slot])
            pltpu.async_copy(t_hbm.at[pl.ds(r0, _PIECE_ROWS), :],
                             tbuf.at[dst, :], tsem.at[slot])

        def wait_piece(slot):
            src = pl.ds(0, _PIECE_ROWS)
            dst = pl.ds(slot * _PIECE_ROWS, _PIECE_ROWS)
            pltpu.make_async_copy(x_hbm.at[src, :], xbuf.at[dst, :],
                                  xsem.at[slot]).wait()
            pltpu.make_async_copy(t_hbm.at[src, :], tbuf.at[dst, :],
                                  tsem.at[slot]).wait()

        def compute_piece(slot):
            for r in range(_PIECE_ROWS):
                row = slot * _PIECE_ROWS + r

                def vec_body(v, _):
                    offs = [pl.multiple_of(16 * (_UNROLL * v + u), 16)
                            for u in range(_UNROLL)]
                    xs = [xbuf[row, pl.ds(o, 16)] for o in offs]
                    ts = [tbuf[row, pl.ds(o, 16)] for o in offs]
                    mbs = [t < 1.5 for t in ts]
                    es = [jnp.exp(-jnp.abs(x)) for x in xs]
                    confs = [1.0 / (1.0 + e) for e in es]
                    tcs = [t * 0.000244140625 for t in ts]
                    paks = [jnp.where(x > 0.0, 1.0 + tc,
                                      1.000244140625 - tc)
                            for x, tc in zip(xs, tcs)]
                    idxs = [lane_off + (c * 15.0).astype(jnp.int32)
                            for c in confs]
                    for u in range(_UNROLL):
                        plsc.addupdate_scatter(hpak[u], [idxs[u]], paks[u],
                                               mask=mbs[u])
                    for u in range(_UNROLL):
                        plsc.addupdate_scatter(hcnf[u], [idxs[u]], confs[u],
                                               mask=mbs[u])
                    return 0

                lax.fori_loop(0, _VPR // _UNROLL, vec_body, 0)

        start_piece(0, 0)
        start_piece(1, 1)

        def pair_body(s, _):
            for slot in range(2):
                p = 2 * s + slot
                wait_piece(slot)
                compute_piece(slot)

                @pl.when(p + 2 < _NPIECE)
                def _prefetch():
                    start_piece(p + 2, slot)
            return 0

        lax.fori_loop(0, _NPIECE // 2, pair_body, 0)
        for u in range(_UNROLL):
            pltpu.sync_copy(hists[u], out_hbm.at[wid, u])

    return sc_ece(logits2d, targets2d)


def kernel(logits, mask, targets):
    tgt_m = jnp.where(mask, targets, 2.0)
    part = _sc_hist_call(logits, tgt_m)
    q = part.reshape(_NW, _UNROLL, 2, 16, 16)
    pak = q[:, :, 0]
    count_g = jnp.floor(pak)
    acc_g = (pak - count_g) * 4096.0
    count = count_g.sum(axis=(0, 1, 2))
    sum_acc = acc_g.sum(axis=(0, 1, 2))
    sum_conf = q[:, :, 1].sum(axis=(0, 1, 2))
    count = count.at[14].add(count[15])[:15]
    sum_conf = sum_conf.at[14].add(sum_conf[15])[:15]
    sum_acc = sum_acc.at[14].add(sum_acc[15])[:15]
    total = jnp.float32(logits.size)
    denom = jnp.maximum(count, 1.0)
    contrib = jnp.where(
        count > 0.0,
        jnp.abs(sum_conf / denom - sum_acc / denom) * (count / total),
        0.0,
    )
    return jnp.sum(contrib, keepdims=True)


# speedup vs baseline: 1.1014x; 1.1014x over previous
import functools

import jax
import jax.numpy as jnp
from jax import lax
from jax.experimental import pallas as pl
from jax.experimental.pallas import tpu as pltpu
from jax.experimental.pallas import tpu_sc as plsc

_ROWS = 8192
_COLS = 2048
_NW = 32
_UNROLL = 8
_ROWS_PER_TILE = _ROWS // _NW
_PIECE_ROWS = 8
_NPIECE = _ROWS_PER_TILE // _PIECE_ROWS
_VPR = _COLS // 16


def _sc_hist_call(logits2d, maskf2d, targets2d):
    mesh = plsc.VectorSubcoreMesh(core_axis_name="c", subcore_axis_name="s")

    @functools.partial(
        pl.kernel,
        out_type=jax.ShapeDtypeStruct((_NW, _UNROLL, 512), jnp.float32),
        mesh=mesh,
        compiler_params=pltpu.CompilerParams(
            needs_layout_passes=False, use_tc_tiling_on_sc=True),
        scratch_types=[
            pltpu.VMEM((2 * _PIECE_ROWS, _COLS), jnp.float32),
            pltpu.VMEM((2 * _PIECE_ROWS, _COLS), jnp.float32),
            pltpu.VMEM((2 * _PIECE_ROWS, _COLS), jnp.float32),
        ] + [pltpu.VMEM((512,), jnp.float32) for _ in range(_UNROLL)] + [
            pltpu.SemaphoreType.DMA((2,)),
            pltpu.SemaphoreType.DMA((2,)),
            pltpu.SemaphoreType.DMA((2,)),
        ],
    )
    def sc_ece(x_hbm, m_hbm, t_hbm, out_hbm, xbuf, tbuf, mbuf, *rest):
        hists = rest[:_UNROLL]
        xsem, tsem, msem = rest[_UNROLL:]
        wid = lax.axis_index("s") * 2 + lax.axis_index("c")
        row_base = wid * _ROWS_PER_TILE

        zeros16 = jnp.zeros((16,), jnp.float32)
        for h in hists:
            for k in range(32):
                h[pl.ds(16 * k, 16)] = zeros16

        lane_off = lax.iota(jnp.int32, 16) * 16
        hpak = [h.at[pl.ds(0, 256)] for h in hists]
        hcnf = [h.at[pl.ds(256, 256)] for h in hists]

        def start_piece(p, slot):
            r0 = pl.multiple_of(row_base + p * _PIECE_ROWS, _PIECE_ROWS)
            dst = pl.ds(slot * _PIECE_ROWS, _PIECE_ROWS)
            pltpu.async_copy(x_hbm.at[pl.ds(r0, _PIECE_ROWS), :],
                             xbuf.at[dst, :], xsem.at[slot])
            pltpu.async_copy(t_hbm.at[pl.ds(r0, _PIECE_ROWS), :],
                             tbuf.at[dst, :], tsem.at[slot])
            pltpu.async_copy(m_hbm.at[pl.ds(r0, _PIECE_ROWS), :],
                             mbuf.at[dst, :], msem.at[slot])

        def wait_piece(slot):
            src = pl.ds(0, _PIECE_ROWS)
            dst = pl.ds(slot * _PIECE_ROWS, _PIECE_ROWS)
            pltpu.make_async_copy(x_hbm.at[src, :], xbuf.at[dst, :],
                                  xsem.at[slot]).wait()
            pltpu.make_async_copy(t_hbm.at[src, :], tbuf.at[dst, :],
                                  tsem.at[slot]).wait()
            pltpu.make_async_copy(m_hbm.at[src, :], mbuf.at[dst, :],
                                  msem.at[slot]).wait()

        def compute_piece(slot):
            for r in range(_PIECE_ROWS):
                row = slot * _PIECE_ROWS + r

                def vec_body(v, _):
                    offs = [pl.multiple_of(16 * (_UNROLL * v + u), 16)
                            for u in range(_UNROLL)]
                    xs = [xbuf[row, pl.ds(o, 16)] for o in offs]
                    ts = [tbuf[row, pl.ds(o, 16)] for o in offs]
                    mbs = [mbuf[row, pl.ds(o, 16)] > 0.5 for o in offs]
                    es = [jnp.exp(-jnp.abs(x)) for x in xs]
                    confs = [1.0 / (1.0 + e) for e in es]
                    tcs = [t * 0.000244140625 for t in ts]
                    paks = [jnp.where(x > 0.0, 1.0 + tc,
                                      1.000244140625 - tc)
                            for x, tc in zip(xs, tcs)]
                    idxs = [lane_off + (c * 15.0).astype(jnp.int32)
                            for c in confs]
                    for u in range(_UNROLL):
                        plsc.addupdate_scatter(hpak[u], [idxs[u]], paks[u],
                                               mask=mbs[u])
                    for u in range(_UNROLL):
                        plsc.addupdate_scatter(hcnf[u], [idxs[u]], confs[u],
                                               mask=mbs[u])
                    return 0

                lax.fori_loop(0, _VPR // _UNROLL, vec_body, 0)

        start_piece(0, 0)
        start_piece(1, 1)

        def pair_body(s, _):
            for slot in range(2):
                p = 2 * s + slot
                wait_piece(slot)
                compute_piece(slot)

                @pl.when(p + 2 < _NPIECE)
                def _prefetch():
                    start_piece(p + 2, slot)
            return 0

        lax.fori_loop(0, _NPIECE // 2, pair_body, 0)
        for u in range(_UNROLL):
            pltpu.sync_copy(hists[u], out_hbm.at[wid, u])

    return sc_ece(logits2d, maskf2d, targets2d)


def kernel(logits, mask, targets):
    mask_f = mask.astype(jnp.float32)
    part = _sc_hist_call(logits, mask_f, targets)
    q = part.reshape(_NW, _UNROLL, 2, 16, 16)
    pak = q[:, :, 0]
    count_g = jnp.floor(pak)
    acc_g = (pak - count_g) * 4096.0
    count = count_g.sum(axis=(0, 1, 2))
    sum_acc = acc_g.sum(axis=(0, 1, 2))
    sum_conf = q[:, :, 1].sum(axis=(0, 1, 2))
    count = count.at[14].add(count[15])[:15]
    sum_conf = sum_conf.at[14].add(sum_conf[15])[:15]
    sum_acc = sum_acc.at[14].add(sum_acc[15])[:15]
    total = jnp.float32(logits.size)
    denom = jnp.maximum(count, 1.0)
    contrib = jnp.where(
        count > 0.0,
        jnp.abs(sum_conf / denom - sum_acc / denom) * (count / total),
        0.0,
    )
    return jnp.sum(contrib, keepdims=True)
